# baseline (speedup 1.0000x reference)
import jax
import jax.numpy as jnp
from jax.experimental import pallas as pl
from jax.experimental.pallas import tpu as pltpu

C = 384
NE = 20
B = 8
H = 24
W = 24
ROWS = H * W
S = 640
G = 32
PBUF = B * S
ADT = jnp.bfloat16


def _body(xt, wmap_t, projw_t, pb, gum, emb, wk, rb, out, p_buf, o_buf):
    i = pl.program_id(0)

    sidx = jax.lax.broadcasted_iota(jnp.int32, (S, 1), 0)
    mask_m = jnp.where(sidx % W == (G % W), 0.0, 1.0).astype(ADT)
    mask_p = jnp.where(sidx % W == ((G - 1) % W), 0.0, 1.0).astype(ADT)

    @pl.when(i == 0)
    def _init():
        p_buf[...] = jnp.zeros((PBUF, C), ADT)
        o_buf[...] = jnp.zeros((PBUF, C), ADT)
        xv = xt[...]
        pooled = jnp.mean(xv, axis=2)
        m = jax.nn.sigmoid(jnp.dot(pooled, wmap_t[...],
                                   preferred_element_type=jnp.float32))
        logits = jnp.dot(m, projw_t[...],
                         preferred_element_type=jnp.float32) + pb[...]
        y = logits + gum[...]
        col = jax.lax.broadcasted_iota(jnp.int32, (B, NE), 1)
        ymax = jnp.max(y, axis=1, keepdims=True)
        amin = jnp.min(jnp.where(y == ymax, col, NE), axis=1, keepdims=True)
        oh = (col == amin).astype(jnp.float32)
        zq = jnp.dot(oh, emb[...], preferred_element_type=jnp.float32)
        for b in range(B):
            vb = jnp.transpose(xv[b]) + zq[b][None, :]
            p_buf[pl.ds(b * S + G, ROWS), :] = vb.astype(ADT)

    def conv(src, dst, j, residual):
        bias = rb[0, j][None, :]

        def chunk(b, carry):
            base = b * S
            slab = src[pl.ds(base, S), :]
            am = jnp.roll(slab, 1, axis=0) * mask_m
            ap = jnp.roll(slab, -1, axis=0) * mask_p
            taps = []
            for t in range(9):
                lo = G + W * (t // 3 - 1)
                sv = (am, slab, ap)[t % 3]
                taps.append(jax.lax.slice(sv, (lo, 0), (lo + ROWS, C)))
            lhs = jnp.concatenate(taps, axis=1)
            acc = jnp.dot(lhs, wk[0, j],
                          preferred_element_type=jnp.float32)
            val = acc + bias
            if residual:
                val = val + dst[pl.ds(base + G, ROWS), :].astype(jnp.float32)
            val = jnp.maximum(val, 0.0)
            dst[pl.ds(base + G, ROWS), :] = val.astype(ADT)
            return carry

        jax.lax.fori_loop(0, B, chunk, 0)

    conv(p_buf, o_buf, 0, False)
    conv(o_buf, p_buf, 1, True)

    @pl.when(i == 2)
    def _fin():
        for b in range(B):
            blk = p_buf[pl.ds(b * S + G, ROWS), :].astype(jnp.float32)
            out[b] = jnp.transpose(blk)


def _build(interpret=False):
    return pl.pallas_call(
        _body,
        grid=(3,),
        in_specs=[
            pl.BlockSpec((B, C, ROWS), lambda i: (0, 0, 0)),
            pl.BlockSpec((C, C), lambda i: (0, 0)),
            pl.BlockSpec((C, NE), lambda i: (0, 0)),
            pl.BlockSpec((1, NE), lambda i: (0, 0)),
            pl.BlockSpec((B, NE), lambda i: (0, 0)),
            pl.BlockSpec((NE, C), lambda i: (0, 0)),
            pl.BlockSpec((1, 2, 9 * C, C), lambda i: (i, 0, 0, 0)),
            pl.BlockSpec((1, 2, C), lambda i: (i, 0, 0)),
        ],
        out_specs=pl.BlockSpec((B, C, ROWS), lambda i: (0, 0, 0)),
        out_shape=jax.ShapeDtypeStruct((B, C, ROWS), jnp.float32),
        scratch_shapes=[pltpu.VMEM((PBUF, C), ADT),
                        pltpu.VMEM((PBUF, C), ADT)],
        compiler_params=pltpu.CompilerParams(
            dimension_semantics=("arbitrary",)),
        interpret=interpret,
    )


def kernel(x, W_map, proj_W, proj_b, embed, res_w, res_b, gumbel):
    xf = x.reshape(B, C, ROWS)
    wmap_t = W_map[:, :, 0, 0].T
    projw_t = proj_W[:, :, 0, 0].T
    pb = proj_b.reshape(1, NE)
    gum = gumbel[:, :, 0, 0]
    wk = jnp.transpose(res_w.astype(ADT), (0, 1, 4, 5, 3, 2)).reshape(3, 2, 9 * C, C)
    out = _build()(xf, wmap_t, projw_t, pb, gum, embed, wk, res_b)
    return out.reshape(B, C, H, W)


# speedup vs baseline: 1.1259x; 1.1259x over previous
import jax
import jax.numpy as jnp
from jax.experimental import pallas as pl
from jax.experimental.pallas import tpu as pltpu

C = 384
NE = 20
B = 8
H = 24
W = 24
ROWS = H * W
S = 640
G = 32
PBUF = B * S
ADT = jnp.bfloat16


def _body(xt, wmap_t, projw_t, pb, gum, emb, wk, rb, out, p_buf, o_buf):
    i = pl.program_id(0)

    sidx = jax.lax.broadcasted_iota(jnp.int32, (S, 1), 0)
    mask_m = jnp.where(sidx % W == (G % W), 0.0, 1.0).astype(ADT)
    mask_p = jnp.where(sidx % W == ((G - 1) % W), 0.0, 1.0).astype(ADT)

    @pl.when(i == 0)
    def _init():
        p_buf[...] = jnp.zeros((PBUF, C), ADT)
        o_buf[...] = jnp.zeros((PBUF, C), ADT)
        xv = xt[...].reshape(B, ROWS, C)
        pooled = jnp.mean(xv, axis=1)
        m = jax.nn.sigmoid(jnp.dot(pooled, wmap_t[...],
                                   preferred_element_type=jnp.float32))
        logits = jnp.dot(m, projw_t[...],
                         preferred_element_type=jnp.float32) + pb[...]
        y = logits + gum[...]
        col = jax.lax.broadcasted_iota(jnp.int32, (B, NE), 1)
        ymax = jnp.max(y, axis=1, keepdims=True)
        amin = jnp.min(jnp.where(y == ymax, col, NE), axis=1, keepdims=True)
        oh = (col == amin).astype(jnp.float32)
        zq = jnp.dot(oh, emb[...], preferred_element_type=jnp.float32)
        v = xv + zq[:, None, :]
        for b in range(B):
            p_buf[pl.ds(b * S + G, ROWS), :] = v[b].astype(ADT)

    def conv(src, dst, j, residual):
        bias = rb[0, j][None, :]

        def chunk(b):
            base = b * S
            slab = src[pl.ds(base, S), :]
            am = jnp.roll(slab, 1, axis=0) * mask_m
            ap = jnp.roll(slab, -1, axis=0) * mask_p
            taps = []
            for t in range(9):
                lo = G + W * (t // 3 - 1)
                sv = (am, slab, ap)[t % 3]
                taps.append(jax.lax.slice(sv, (lo, 0), (lo + ROWS, C)))
            lhs = jnp.concatenate(taps, axis=1)
            acc = jnp.dot(lhs, wk[0, j],
                          preferred_element_type=jnp.float32)
            val = acc + bias
            if residual:
                val = val + dst[pl.ds(base + G, ROWS), :].astype(jnp.float32)
            val = jnp.maximum(val, 0.0)
            dst[pl.ds(base + G, ROWS), :] = val.astype(ADT)

        for b in range(B):
            chunk(b)

    conv(p_buf, o_buf, 0, False)
    conv(o_buf, p_buf, 1, True)

    @pl.when(i == 2)
    def _fin():
        for b in range(B):
            blk = p_buf[pl.ds(b * S + G, ROWS), :].reshape(H, W, C)
            out[b] = blk.astype(jnp.float32)


def _build(interpret=False):
    return pl.pallas_call(
        _body,
        grid=(3,),
        in_specs=[
            pl.BlockSpec((B, H, W, C), lambda i: (0, 0, 0, 0)),
            pl.BlockSpec((C, C), lambda i: (0, 0)),
            pl.BlockSpec((C, NE), lambda i: (0, 0)),
            pl.BlockSpec((1, NE), lambda i: (0, 0)),
            pl.BlockSpec((B, NE), lambda i: (0, 0)),
            pl.BlockSpec((NE, C), lambda i: (0, 0)),
            pl.BlockSpec((1, 2, 9 * C, C), lambda i: (i, 0, 0, 0)),
            pl.BlockSpec((1, 2, C), lambda i: (i, 0, 0)),
        ],
        out_specs=pl.BlockSpec((B, H, W, C), lambda i: (0, 0, 0, 0)),
        out_shape=jax.ShapeDtypeStruct((B, H, W, C), jnp.float32),
        scratch_shapes=[pltpu.VMEM((PBUF, C), ADT),
                        pltpu.VMEM((PBUF, C), ADT)],
        compiler_params=pltpu.CompilerParams(
            dimension_semantics=("arbitrary",)),
        interpret=interpret,
    )


def kernel(x, W_map, proj_W, proj_b, embed, res_w, res_b, gumbel):
    xt = jnp.transpose(x, (0, 2, 3, 1))
    wmap_t = W_map[:, :, 0, 0].T
    projw_t = proj_W[:, :, 0, 0].T
    pb = proj_b.reshape(1, NE)
    gum = gumbel[:, :, 0, 0]
    wk = jnp.transpose(res_w.astype(ADT), (0, 1, 4, 5, 3, 2)).reshape(3, 2, 9 * C, C)
    out = _build()(xt, wmap_t, projw_t, pb, gum, embed, wk, res_b)
    return jnp.transpose(out, (0, 3, 1, 2))
